# Initial kernel scaffold; baseline (speedup 1.0000x reference)
import functools

import jax
import jax.numpy as jnp
from jax import lax
from jax.experimental import pallas as pl
from jax.experimental.pallas import tpu as pltpu
from jax.experimental.pallas import tpu_sc as plsc

VOCAB = 100000
EMBED = 64
BATCH = 4096
SEQ = 200

_NC = 2
_NS = 16
_NW = _NC * _NS
_ROWS_PER_W = BATCH // _NW
_CHUNK0 = 128
_CHUNK1 = SEQ - _CHUNK0


@functools.partial(
    pl.kernel,
    out_type=jax.ShapeDtypeStruct((BATCH, EMBED), jnp.float32),
    mesh=plsc.VectorSubcoreMesh(core_axis_name="c", subcore_axis_name="s"),
    scratch_types=[
        pltpu.VMEM((_ROWS_PER_W, SEQ), jnp.int32),
        pltpu.VMEM((SEQ, EMBED), jnp.float32),
        pltpu.VMEM((_ROWS_PER_W, EMBED), jnp.float32),
        pltpu.SemaphoreType.DMA,
    ],
)
def _pool_sc(x_hbm, table_hbm, pooled_hbm, idx_v, rows_v, pooled_v, sem):
    wid = lax.axis_index("s") * _NC + lax.axis_index("c")
    base = wid * _ROWS_PER_W

    pltpu.sync_copy(x_hbm.at[pl.ds(base, _ROWS_PER_W)], idx_v)

    def row_body(i, carry):
        c0 = pltpu.async_copy(
            table_hbm.at[idx_v.at[i, pl.ds(0, _CHUNK0)]],
            rows_v.at[pl.ds(0, _CHUNK0)], sem)
        c1 = pltpu.async_copy(
            table_hbm.at[idx_v.at[i, pl.ds(_CHUNK0, _CHUNK1)]],
            rows_v.at[pl.ds(_CHUNK0, _CHUNK1)], sem)
        c0.wait()
        c1.wait()

        zeros = jnp.zeros((16,), jnp.float32)

        def seq_body(r, accs):
            a0, a1, a2, a3 = accs
            return (a0 + rows_v[r, pl.ds(0, 16)],
                    a1 + rows_v[r, pl.ds(16, 16)],
                    a2 + rows_v[r, pl.ds(32, 16)],
                    a3 + rows_v[r, pl.ds(48, 16)])

        a0, a1, a2, a3 = lax.fori_loop(0, SEQ, seq_body,
                                       (zeros, zeros, zeros, zeros))
        scale = jnp.float32(1.0 / SEQ)
        pooled_v[i, pl.ds(0, 16)] = a0 * scale
        pooled_v[i, pl.ds(16, 16)] = a1 * scale
        pooled_v[i, pl.ds(32, 16)] = a2 * scale
        pooled_v[i, pl.ds(48, 16)] = a3 * scale
        return carry

    lax.fori_loop(0, _ROWS_PER_W, row_body, 0)
    pltpu.sync_copy(pooled_v, pooled_hbm.at[pl.ds(base, _ROWS_PER_W)])


_TB = 64


def _mm_softmax_body(pooled_ref, w_ref, b_ref, out_ref):
    logits = lax.dot_general(
        pooled_ref[...], w_ref[...],
        dimension_numbers=(((1,), (1,)), ((), ())),
        preferred_element_type=jnp.float32)
    logits = logits + b_ref[...]
    m = jnp.max(logits, axis=1, keepdims=True)
    e = jnp.exp(logits - m)
    s = jnp.sum(e, axis=1, keepdims=True)
    out_ref[...] = e / s


def _mm_softmax(pooled, W, b2):
    return pl.pallas_call(
        _mm_softmax_body,
        grid=(BATCH // _TB,),
        in_specs=[
            pl.BlockSpec((_TB, EMBED), lambda i: (i, 0)),
            pl.BlockSpec((VOCAB, EMBED), lambda i: (0, 0)),
            pl.BlockSpec((1, VOCAB), lambda i: (0, 0)),
        ],
        out_specs=pl.BlockSpec((_TB, VOCAB), lambda i: (i, 0)),
        out_shape=jax.ShapeDtypeStruct((BATCH, VOCAB), jnp.float32),
    )(pooled, W, b2)


def kernel(x, embed_table, W, b):
    pooled = _pool_sc(x, embed_table)
    return _mm_softmax(pooled, W, b.reshape(1, VOCAB))


# speedup vs baseline: 2.0116x; 2.0116x over previous
import functools

import jax
import jax.numpy as jnp
from jax import lax
from jax.experimental import pallas as pl
from jax.experimental.pallas import tpu as pltpu
from jax.experimental.pallas import tpu_sc as plsc

VOCAB = 100000
EMBED = 64
BATCH = 4096
SEQ = 200

_NC = 2
_NS = 16
_NW = _NC * _NS
_ROWS_PER_W = BATCH // _NW
_CHUNK0 = 128
_CHUNK1 = SEQ - _CHUNK0


@functools.partial(
    pl.kernel,
    out_type=jax.ShapeDtypeStruct((BATCH, EMBED), jnp.float32),
    mesh=plsc.VectorSubcoreMesh(core_axis_name="c", subcore_axis_name="s"),
    compiler_params=pltpu.CompilerParams(use_tc_tiling_on_sc=False),
    scratch_types=[
        pltpu.VMEM((_ROWS_PER_W, SEQ), jnp.int32),
        pltpu.VMEM((SEQ, EMBED), jnp.float32),
        pltpu.VMEM((_ROWS_PER_W, EMBED), jnp.float32),
        pltpu.SemaphoreType.DMA,
    ],
)
def _pool_sc(x_hbm, table_hbm, pooled_hbm, idx_v, rows_v, pooled_v, sem):
    wid = lax.axis_index("s") * _NC + lax.axis_index("c")
    base = wid * _ROWS_PER_W

    pltpu.sync_copy(x_hbm.at[pl.ds(base, _ROWS_PER_W)], idx_v)

    def row_body(i, carry):
        c0 = pltpu.async_copy(
            table_hbm.at[idx_v.at[i, pl.ds(0, _CHUNK0)]],
            rows_v.at[pl.ds(0, _CHUNK0)], sem)
        c1 = pltpu.async_copy(
            table_hbm.at[idx_v.at[i, pl.ds(_CHUNK0, _CHUNK1)]],
            rows_v.at[pl.ds(_CHUNK0, _CHUNK1)], sem)
        c0.wait()
        c1.wait()

        zeros = jnp.zeros((16,), jnp.float32)

        def seq_body(r, accs):
            a0, a1, a2, a3 = accs
            return (a0 + rows_v[r, pl.ds(0, 16)],
                    a1 + rows_v[r, pl.ds(16, 16)],
                    a2 + rows_v[r, pl.ds(32, 16)],
                    a3 + rows_v[r, pl.ds(48, 16)])

        a0, a1, a2, a3 = lax.fori_loop(0, SEQ, seq_body,
                                       (zeros, zeros, zeros, zeros))
        scale = jnp.float32(1.0 / SEQ)
        pooled_v[i, pl.ds(0, 16)] = a0 * scale
        pooled_v[i, pl.ds(16, 16)] = a1 * scale
        pooled_v[i, pl.ds(32, 16)] = a2 * scale
        pooled_v[i, pl.ds(48, 16)] = a3 * scale
        return carry

    lax.fori_loop(0, _ROWS_PER_W, row_body, 0)
    pltpu.sync_copy(pooled_v, pooled_hbm.at[pl.ds(base, _ROWS_PER_W)])


_TB = 32
_CH = 12544
_CHUNKS = [(o, min(_CH, VOCAB - o)) for o in range(0, VOCAB, _CH)]


def _mm_softmax_body(pooled_ref, wt_ref, b_ref, out_ref):
    p = pooled_ref[...]
    for off, sz in _CHUNKS:
        lt = lax.dot_general(
            p, wt_ref[:, pl.ds(off, sz)],
            dimension_numbers=(((1,), (0,)), ((), ())),
            preferred_element_type=jnp.float32)
        out_ref[:, pl.ds(off, sz)] = lt + b_ref[:, pl.ds(off, sz)]
    m = jnp.full((_TB, 1), -jnp.inf, jnp.float32)
    for off, sz in _CHUNKS:
        m = jnp.maximum(m, jnp.max(out_ref[:, pl.ds(off, sz)], axis=1,
                                   keepdims=True))
    s = jnp.zeros((_TB, 1), jnp.float32)
    for off, sz in _CHUNKS:
        e = jnp.exp(out_ref[:, pl.ds(off, sz)] - m)
        out_ref[:, pl.ds(off, sz)] = e
        s = s + jnp.sum(e, axis=1, keepdims=True)
    inv = 1.0 / s
    for off, sz in _CHUNKS:
        out_ref[:, pl.ds(off, sz)] = out_ref[:, pl.ds(off, sz)] * inv


def _mm_softmax(pooled, Wt, b2):
    return pl.pallas_call(
        _mm_softmax_body,
        grid=(BATCH // _TB,),
        in_specs=[
            pl.BlockSpec((_TB, EMBED), lambda i: (i, 0)),
            pl.BlockSpec((EMBED, VOCAB), lambda i: (0, 0)),
            pl.BlockSpec((1, VOCAB), lambda i: (0, 0)),
        ],
        out_specs=pl.BlockSpec((_TB, VOCAB), lambda i: (i, 0)),
        out_shape=jax.ShapeDtypeStruct((BATCH, VOCAB), jnp.float32),
    )(pooled, Wt, b2)


def kernel(x, embed_table, W, b):
    pooled = _pool_sc(x, embed_table)
    Wt = W.T.astype(jnp.bfloat16)
    return _mm_softmax(pooled.astype(jnp.bfloat16), Wt, b.reshape(1, VOCAB))


# speedup vs baseline: 3.0767x; 1.5295x over previous
import functools

import jax
import jax.numpy as jnp
from jax import lax
from jax.experimental import pallas as pl
from jax.experimental.pallas import tpu as pltpu
from jax.experimental.pallas import tpu_sc as plsc

VOCAB = 100000
EMBED = 64
BATCH = 4096
SEQ = 200

_NC = 2
_NS = 16
_NW = _NC * _NS
_ROWS_PER_W = BATCH // _NW
_CHUNK0 = 128
_CHUNK1 = SEQ - _CHUNK0


@functools.partial(
    pl.kernel,
    out_type=jax.ShapeDtypeStruct((BATCH, EMBED), jnp.float32),
    mesh=plsc.VectorSubcoreMesh(core_axis_name="c", subcore_axis_name="s"),
    compiler_params=pltpu.CompilerParams(use_tc_tiling_on_sc=False),
    scratch_types=[
        pltpu.VMEM((_ROWS_PER_W, SEQ), jnp.int32),
        pltpu.VMEM((SEQ, EMBED), jnp.float32),
        pltpu.VMEM((_ROWS_PER_W, EMBED), jnp.float32),
        pltpu.SemaphoreType.DMA,
    ],
)
def _pool_sc(x_hbm, table_hbm, pooled_hbm, idx_v, rows_v, pooled_v, sem):
    wid = lax.axis_index("s") * _NC + lax.axis_index("c")
    base = wid * _ROWS_PER_W

    pltpu.sync_copy(x_hbm.at[pl.ds(base, _ROWS_PER_W)], idx_v)

    def row_body(i, carry):
        c0 = pltpu.async_copy(
            table_hbm.at[idx_v.at[i, pl.ds(0, _CHUNK0)]],
            rows_v.at[pl.ds(0, _CHUNK0)], sem)
        c1 = pltpu.async_copy(
            table_hbm.at[idx_v.at[i, pl.ds(_CHUNK0, _CHUNK1)]],
            rows_v.at[pl.ds(_CHUNK0, _CHUNK1)], sem)
        c0.wait()
        c1.wait()

        zeros = jnp.zeros((16,), jnp.float32)

        def seq_body(r, accs):
            a0, a1, a2, a3 = accs
            return (a0 + rows_v[r, pl.ds(0, 16)],
                    a1 + rows_v[r, pl.ds(16, 16)],
                    a2 + rows_v[r, pl.ds(32, 16)],
                    a3 + rows_v[r, pl.ds(48, 16)])

        a0, a1, a2, a3 = lax.fori_loop(0, SEQ, seq_body,
                                       (zeros, zeros, zeros, zeros))
        scale = jnp.float32(1.0 / SEQ)
        pooled_v[i, pl.ds(0, 16)] = a0 * scale
        pooled_v[i, pl.ds(16, 16)] = a1 * scale
        pooled_v[i, pl.ds(32, 16)] = a2 * scale
        pooled_v[i, pl.ds(48, 16)] = a3 * scale
        return carry

    lax.fori_loop(0, _ROWS_PER_W, row_body, 0)
    pltpu.sync_copy(pooled_v, pooled_hbm.at[pl.ds(base, _ROWS_PER_W)])


_VC = 20000
_VT = VOCAB // _VC
_BT = 128
_KA = 128


def _stats_body(w_ref, p_ref, stats_ref, m_scr, s_scr):
    vc = pl.program_id(0)
    bt = pl.program_id(1)
    sl = pl.ds(bt * _BT, _BT)
    lt = lax.dot_general(
        w_ref[...], p_ref[...],
        dimension_numbers=(((1,), (0,)), ((), ())),
        preferred_element_type=jnp.float32)
    local_m = jnp.max(lt, axis=0, keepdims=True)
    first = vc == 0
    m_old = jnp.where(first, -jnp.inf, m_scr[:, sl])
    s_old = jnp.where(first, 0.0, s_scr[:, sl])
    m_new = jnp.maximum(m_old, local_m)
    s_new = (s_old * jnp.exp(m_old - m_new)
             + jnp.sum(jnp.exp(lt - m_new), axis=0, keepdims=True))
    m_scr[:, sl] = m_new
    s_scr[:, sl] = s_new
    stats_ref[0:1, :] = m_new
    stats_ref[1:2, :] = 1.0 / s_new


def _stats(W_aug, pT_aug):
    return pl.pallas_call(
        _stats_body,
        grid=(_VT, BATCH // _BT),
        in_specs=[
            pl.BlockSpec((_VC, _KA), lambda v, b: (v, 0)),
            pl.BlockSpec((_KA, _BT), lambda v, b: (0, b)),
        ],
        out_specs=pl.BlockSpec((2, _BT), lambda v, b: (0, b)),
        out_shape=jax.ShapeDtypeStruct((2, BATCH), jnp.float32),
        scratch_shapes=[
            pltpu.VMEM((1, BATCH), jnp.float32),
            pltpu.VMEM((1, BATCH), jnp.float32),
        ],
    )(W_aug, pT_aug)


def _write_body(w_ref, p_ref, stats_ref, out_ref):
    lt = lax.dot_general(
        w_ref[...], p_ref[...],
        dimension_numbers=(((1,), (0,)), ((), ())),
        preferred_element_type=jnp.float32)
    m = stats_ref[0:1, :]
    inv = stats_ref[1:2, :]
    out_ref[...] = jnp.exp(lt - m) * inv


def _write(W_aug, pT_aug, stats):
    return pl.pallas_call(
        _write_body,
        grid=(_VT, BATCH // _BT),
        in_specs=[
            pl.BlockSpec((_VC, _KA), lambda v, b: (v, 0)),
            pl.BlockSpec((_KA, _BT), lambda v, b: (0, b)),
            pl.BlockSpec((2, _BT), lambda v, b: (0, b)),
        ],
        out_specs=pl.BlockSpec((_VC, _BT), lambda v, b: (v, b)),
        out_shape=jax.ShapeDtypeStruct((VOCAB, BATCH), jnp.float32),
    )(W_aug, pT_aug, stats)


def kernel(x, embed_table, W, b):
    pooled = _pool_sc(x, embed_table)
    W_aug = jnp.pad(
        jnp.concatenate([W, b[:, None]], axis=1),
        ((0, 0), (0, _KA - EMBED - 1))).astype(jnp.bfloat16)
    pT_aug = jnp.concatenate(
        [pooled.T,
         jnp.ones((1, BATCH), jnp.float32),
         jnp.zeros((_KA - EMBED - 1, BATCH), jnp.float32)],
        axis=0).astype(jnp.bfloat16)
    stats = _stats(W_aug, pT_aug)
    outT = _write(W_aug, pT_aug, stats)
    return outT.T


# speedup vs baseline: 3.8451x; 1.2498x over previous
import functools

import jax
import jax.numpy as jnp
from jax import lax
from jax.experimental import pallas as pl
from jax.experimental.pallas import tpu as pltpu
from jax.experimental.pallas import tpu_sc as plsc

VOCAB = 100000
EMBED = 64
BATCH = 4096
SEQ = 200

_NC = 2
_NS = 16
_NW = _NC * _NS
_ROWS_PER_W = BATCH // _NW
_CHUNK0 = 128
_CHUNK1 = SEQ - _CHUNK0


@functools.partial(
    pl.kernel,
    out_type=jax.ShapeDtypeStruct((BATCH, EMBED), jnp.float32),
    mesh=plsc.VectorSubcoreMesh(core_axis_name="c", subcore_axis_name="s"),
    compiler_params=pltpu.CompilerParams(use_tc_tiling_on_sc=False),
    scratch_types=[
        pltpu.VMEM((_ROWS_PER_W, SEQ), jnp.int32),
        pltpu.VMEM((SEQ, EMBED), jnp.float32),
        pltpu.VMEM((_ROWS_PER_W, EMBED), jnp.float32),
        pltpu.SemaphoreType.DMA,
    ],
)
def _pool_sc(x_hbm, table_hbm, pooled_hbm, idx_v, rows_v, pooled_v, sem):
    wid = lax.axis_index("s") * _NC + lax.axis_index("c")
    base = wid * _ROWS_PER_W

    pltpu.sync_copy(x_hbm.at[pl.ds(base, _ROWS_PER_W)], idx_v)

    def row_body(i, carry):
        c0 = pltpu.async_copy(
            table_hbm.at[idx_v.at[i, pl.ds(0, _CHUNK0)]],
            rows_v.at[pl.ds(0, _CHUNK0)], sem)
        c1 = pltpu.async_copy(
            table_hbm.at[idx_v.at[i, pl.ds(_CHUNK0, _CHUNK1)]],
            rows_v.at[pl.ds(_CHUNK0, _CHUNK1)], sem)
        c0.wait()
        c1.wait()

        zeros = jnp.zeros((16,), jnp.float32)

        def seq_body(r, accs):
            a0, a1, a2, a3 = accs
            return (a0 + rows_v[r, pl.ds(0, 16)],
                    a1 + rows_v[r, pl.ds(16, 16)],
                    a2 + rows_v[r, pl.ds(32, 16)],
                    a3 + rows_v[r, pl.ds(48, 16)])

        a0, a1, a2, a3 = lax.fori_loop(0, SEQ, seq_body,
                                       (zeros, zeros, zeros, zeros))
        scale = jnp.float32(1.0 / SEQ)
        pooled_v[i, pl.ds(0, 16)] = a0 * scale
        pooled_v[i, pl.ds(16, 16)] = a1 * scale
        pooled_v[i, pl.ds(32, 16)] = a2 * scale
        pooled_v[i, pl.ds(48, 16)] = a3 * scale
        return carry

    lax.fori_loop(0, _ROWS_PER_W, row_body, 0)
    pltpu.sync_copy(pooled_v, pooled_hbm.at[pl.ds(base, _ROWS_PER_W)])


_VC = 20000
_VT = VOCAB // _VC
_BT = 128
_KA = 128


_SBT = 256
_SUB = 5000


def _stats_body(w_ref, p_ref, stats_ref, m_scr, s_scr):
    vc = pl.program_id(0)
    bt = pl.program_id(1)
    sl = pl.ds(bt * _SBT, _SBT)
    first = vc == 0
    m_run = jnp.where(first, -jnp.inf, m_scr[:, sl])
    s_run = jnp.where(first, 0.0, s_scr[:, sl])
    p = p_ref[...]
    for off in range(0, _VC, _SUB):
        lt = lax.dot_general(
            w_ref[pl.ds(off, _SUB), :], p,
            dimension_numbers=(((1,), (0,)), ((), ())),
            preferred_element_type=jnp.float32)
        m_new = jnp.maximum(m_run, jnp.max(lt, axis=0, keepdims=True))
        s_run = (s_run * jnp.exp(m_run - m_new)
                 + jnp.sum(jnp.exp(lt - m_new), axis=0, keepdims=True))
        m_run = m_new
    m_scr[:, sl] = m_run
    s_scr[:, sl] = s_run
    stats_ref[0:1, :] = m_run
    stats_ref[1:2, :] = 1.0 / s_run


def _stats(W_aug, pT_aug):
    return pl.pallas_call(
        _stats_body,
        grid=(_VT, BATCH // _SBT),
        in_specs=[
            pl.BlockSpec((_VC, _KA), lambda v, b: (v, 0)),
            pl.BlockSpec((_KA, _SBT), lambda v, b: (0, b)),
        ],
        out_specs=pl.BlockSpec((2, _SBT), lambda v, b: (0, b)),
        out_shape=jax.ShapeDtypeStruct((2, BATCH), jnp.float32),
        scratch_shapes=[
            pltpu.VMEM((1, BATCH), jnp.float32),
            pltpu.VMEM((1, BATCH), jnp.float32),
        ],
    )(W_aug, pT_aug)


def _write_body(w_ref, p_ref, stats_ref, out_ref):
    lt = lax.dot_general(
        w_ref[...], p_ref[...],
        dimension_numbers=(((1,), (0,)), ((), ())),
        preferred_element_type=jnp.float32)
    m = stats_ref[0:1, :]
    inv = stats_ref[1:2, :]
    out_ref[...] = jnp.exp(lt - m) * inv


def _write(W_aug, pT_aug, stats):
    return pl.pallas_call(
        _write_body,
        grid=(_VT, BATCH // _BT),
        in_specs=[
            pl.BlockSpec((_VC, _KA), lambda v, b: (v, 0)),
            pl.BlockSpec((_KA, _BT), lambda v, b: (0, b)),
            pl.BlockSpec((2, _BT), lambda v, b: (0, b)),
        ],
        out_specs=pl.BlockSpec((_VC, _BT), lambda v, b: (v, b)),
        out_shape=jax.ShapeDtypeStruct((VOCAB, BATCH), jnp.float32),
    )(W_aug, pT_aug, stats)


def kernel(x, embed_table, W, b):
    pooled = _pool_sc(x, embed_table)
    W_aug = jnp.concatenate(
        [W.astype(jnp.bfloat16),
         b[:, None].astype(jnp.bfloat16),
         jnp.zeros((VOCAB, _KA - EMBED - 1), jnp.bfloat16)], axis=1)
    pT_aug = jnp.concatenate(
        [pooled.T.astype(jnp.bfloat16),
         jnp.ones((1, BATCH), jnp.bfloat16),
         jnp.zeros((_KA - EMBED - 1, BATCH), jnp.bfloat16)], axis=0)
    stats = _stats(W_aug, pT_aug)
    outT = _write(W_aug, pT_aug, stats)
    return outT.T


# speedup vs baseline: 4.3441x; 1.1298x over previous
import functools

import jax
import jax.numpy as jnp
from jax import lax
from jax.experimental import pallas as pl
from jax.experimental.pallas import tpu as pltpu
from jax.experimental.pallas import tpu_sc as plsc

VOCAB = 100000
EMBED = 64
BATCH = 4096
SEQ = 200

_NC = 2
_NS = 16
_NW = _NC * _NS
_ROWS_PER_W = BATCH // _NW
_CHUNK0 = 128
_CHUNK1 = SEQ - _CHUNK0


_NBUF = 4


@functools.partial(
    pl.kernel,
    out_type=jax.ShapeDtypeStruct((BATCH, EMBED), jnp.float32),
    mesh=plsc.VectorSubcoreMesh(core_axis_name="c", subcore_axis_name="s"),
    compiler_params=pltpu.CompilerParams(use_tc_tiling_on_sc=False),
    scratch_types=[
        pltpu.VMEM((_ROWS_PER_W, SEQ), jnp.int32),
        pltpu.VMEM((_NBUF, SEQ, EMBED), jnp.float32),
        pltpu.VMEM((_ROWS_PER_W, EMBED), jnp.float32),
    ] + [pltpu.SemaphoreType.DMA] * _NBUF,
)
def _pool_sc(x_hbm, table_hbm, pooled_hbm, idx_v, rows_v, pooled_v, *sems):
    wid = lax.axis_index("s") * _NC + lax.axis_index("c")
    base = wid * _ROWS_PER_W

    pltpu.sync_copy(x_hbm.at[pl.ds(base, _ROWS_PER_W)], idx_v)

    def fire(row, buf):
        pltpu.async_copy(
            table_hbm.at[idx_v.at[row, pl.ds(0, _CHUNK0)]],
            rows_v.at[buf, pl.ds(0, _CHUNK0)], sems[buf])
        pltpu.async_copy(
            table_hbm.at[idx_v.at[row, pl.ds(_CHUNK0, _CHUNK1)]],
            rows_v.at[buf, pl.ds(_CHUNK0, _CHUNK1)], sems[buf])

    def drain(buf):
        pltpu.make_async_copy(
            table_hbm.at[pl.ds(0, SEQ)], rows_v.at[buf], sems[buf]).wait()

    for b in range(_NBUF - 1):
        fire(jnp.int32(b), b)

    def quad_body(p, carry):
        r0 = p * _NBUF
        for b in range(_NBUF):
            row = r0 + b
            nxt = row + _NBUF - 1

            @pl.when(nxt < _ROWS_PER_W)
            def _():
                fire(nxt, (b + _NBUF - 1) % _NBUF)

            drain(b)
            zeros = jnp.zeros((16,), jnp.float32)

            def seq_body(r, accs, _b=b):
                a0, a1, a2, a3 = accs
                r2 = 2 * r
                a0 = a0 + rows_v[_b, r2, pl.ds(0, 16)]
                a1 = a1 + rows_v[_b, r2, pl.ds(16, 16)]
                a2 = a2 + rows_v[_b, r2, pl.ds(32, 16)]
                a3 = a3 + rows_v[_b, r2, pl.ds(48, 16)]
                a0 = a0 + rows_v[_b, r2 + 1, pl.ds(0, 16)]
                a1 = a1 + rows_v[_b, r2 + 1, pl.ds(16, 16)]
                a2 = a2 + rows_v[_b, r2 + 1, pl.ds(32, 16)]
                a3 = a3 + rows_v[_b, r2 + 1, pl.ds(48, 16)]
                return (a0, a1, a2, a3)

            a0, a1, a2, a3 = lax.fori_loop(0, SEQ // 2, seq_body,
                                           (zeros, zeros, zeros, zeros))
            scale = jnp.float32(1.0 / SEQ)
            pooled_v[row, pl.ds(0, 16)] = a0 * scale
            pooled_v[row, pl.ds(16, 16)] = a1 * scale
            pooled_v[row, pl.ds(32, 16)] = a2 * scale
            pooled_v[row, pl.ds(48, 16)] = a3 * scale
        return carry

    lax.fori_loop(0, _ROWS_PER_W // _NBUF, quad_body, 0)
    pltpu.sync_copy(pooled_v, pooled_hbm.at[pl.ds(base, _ROWS_PER_W)])


_VC = 20000
_VT = VOCAB // _VC
_BT = 128
_KA = 128


_SBT = 256
_SUB = 5000


def _stats_body(w_ref, p_ref, stats_ref, m_scr, s_scr):
    vc = pl.program_id(0)
    bt = pl.program_id(1)
    sl = pl.ds(bt * _SBT, _SBT)
    first = vc == 0
    m_run = jnp.where(first, -jnp.inf, m_scr[:, sl])
    s_run = jnp.where(first, 0.0, s_scr[:, sl])
    p = p_ref[...]
    for off in range(0, _VC, _SUB):
        lt = lax.dot_general(
            w_ref[pl.ds(off, _SUB), :], p,
            dimension_numbers=(((1,), (0,)), ((), ())),
            preferred_element_type=jnp.float32)
        m_new = jnp.maximum(m_run, jnp.max(lt, axis=0, keepdims=True))
        s_run = (s_run * jnp.exp(m_run - m_new)
                 + jnp.sum(jnp.exp(lt - m_new), axis=0, keepdims=True))
        m_run = m_new
    m_scr[:, sl] = m_run
    s_scr[:, sl] = s_run
    stats_ref[0:1, :] = m_run
    stats_ref[1:2, :] = 1.0 / s_run


def _stats(W_aug, pT_aug):
    return pl.pallas_call(
        _stats_body,
        grid=(_VT, BATCH // _SBT),
        in_specs=[
            pl.BlockSpec((_VC, _KA), lambda v, b: (v, 0)),
            pl.BlockSpec((_KA, _SBT), lambda v, b: (0, b)),
        ],
        out_specs=pl.BlockSpec((2, _SBT), lambda v, b: (0, b)),
        out_shape=jax.ShapeDtypeStruct((2, BATCH), jnp.float32),
        scratch_shapes=[
            pltpu.VMEM((1, BATCH), jnp.float32),
            pltpu.VMEM((1, BATCH), jnp.float32),
        ],
    )(W_aug, pT_aug)


def _write_body(w_ref, p_ref, stats_ref, out_ref):
    lt = lax.dot_general(
        w_ref[...], p_ref[...],
        dimension_numbers=(((1,), (0,)), ((), ())),
        preferred_element_type=jnp.float32)
    m = stats_ref[0:1, :]
    inv = stats_ref[1:2, :]
    out_ref[...] = jnp.exp(lt - m) * inv


def _write(W_aug, pT_aug, stats):
    return pl.pallas_call(
        _write_body,
        grid=(_VT, BATCH // _BT),
        in_specs=[
            pl.BlockSpec((_VC, _KA), lambda v, b: (v, 0)),
            pl.BlockSpec((_KA, _BT), lambda v, b: (0, b)),
            pl.BlockSpec((2, _BT), lambda v, b: (0, b)),
        ],
        out_specs=pl.BlockSpec((_VC, _BT), lambda v, b: (v, b)),
        out_shape=jax.ShapeDtypeStruct((VOCAB, BATCH), jnp.float32),
    )(W_aug, pT_aug, stats)


def kernel(x, embed_table, W, b):
    pooled = _pool_sc(x, embed_table)
    W_aug = jnp.concatenate(
        [W.astype(jnp.bfloat16),
         b[:, None].astype(jnp.bfloat16),
         jnp.zeros((VOCAB, _KA - EMBED - 1), jnp.bfloat16)], axis=1)
    pT_aug = jnp.concatenate(
        [pooled.T.astype(jnp.bfloat16),
         jnp.ones((1, BATCH), jnp.bfloat16),
         jnp.zeros((_KA - EMBED - 1, BATCH), jnp.bfloat16)], axis=0)
    stats = _stats(W_aug, pT_aug)
    outT = _write(W_aug, pT_aug, stats)
    return outT.T


# speedup vs baseline: 4.4256x; 1.0188x over previous
import functools

import jax
import jax.numpy as jnp
from jax import lax
from jax.experimental import pallas as pl
from jax.experimental.pallas import tpu as pltpu
from jax.experimental.pallas import tpu_sc as plsc

VOCAB = 100000
EMBED = 64
BATCH = 4096
SEQ = 200

_NC = 2
_NS = 16
_NW = _NC * _NS
_ROWS_PER_W = BATCH // _NW
_CHUNK0 = 128
_CHUNK1 = SEQ - _CHUNK0


_NBUF = 4


@functools.partial(
    pl.kernel,
    out_type=jax.ShapeDtypeStruct((BATCH, EMBED), jnp.float32),
    mesh=plsc.VectorSubcoreMesh(core_axis_name="c", subcore_axis_name="s"),
    compiler_params=pltpu.CompilerParams(use_tc_tiling_on_sc=False),
    scratch_types=[
        pltpu.VMEM((_ROWS_PER_W, SEQ), jnp.int32),
        pltpu.VMEM((_NBUF, SEQ, EMBED), jnp.float32),
        pltpu.VMEM((_ROWS_PER_W, EMBED), jnp.float32),
    ] + [pltpu.SemaphoreType.DMA] * _NBUF,
)
def _pool_sc(x_hbm, table_hbm, pooled_hbm, idx_v, rows_v, pooled_v, *sems):
    wid = lax.axis_index("s") * _NC + lax.axis_index("c")
    base = wid * _ROWS_PER_W

    pltpu.sync_copy(x_hbm.at[pl.ds(base, _ROWS_PER_W)], idx_v)

    def fire(row, buf):
        pltpu.async_copy(
            table_hbm.at[idx_v.at[row, pl.ds(0, _CHUNK0)]],
            rows_v.at[buf, pl.ds(0, _CHUNK0)], sems[buf])
        pltpu.async_copy(
            table_hbm.at[idx_v.at[row, pl.ds(_CHUNK0, _CHUNK1)]],
            rows_v.at[buf, pl.ds(_CHUNK0, _CHUNK1)], sems[buf])

    def drain(buf):
        pltpu.make_async_copy(
            table_hbm.at[pl.ds(0, SEQ)], rows_v.at[buf], sems[buf]).wait()

    for b in range(_NBUF - 1):
        fire(jnp.int32(b), b)

    def quad_body(p, carry):
        r0 = p * _NBUF
        for b in range(_NBUF):
            row = r0 + b
            nxt = row + _NBUF - 1

            @pl.when(nxt < _ROWS_PER_W)
            def _():
                fire(nxt, (b + _NBUF - 1) % _NBUF)

            drain(b)
            zeros = jnp.zeros((16,), jnp.float32)

            def seq_body(r, accs, _b=b):
                a0, a1, a2, a3 = accs
                r2 = 2 * r
                a0 = a0 + rows_v[_b, r2, pl.ds(0, 16)]
                a1 = a1 + rows_v[_b, r2, pl.ds(16, 16)]
                a2 = a2 + rows_v[_b, r2, pl.ds(32, 16)]
                a3 = a3 + rows_v[_b, r2, pl.ds(48, 16)]
                a0 = a0 + rows_v[_b, r2 + 1, pl.ds(0, 16)]
                a1 = a1 + rows_v[_b, r2 + 1, pl.ds(16, 16)]
                a2 = a2 + rows_v[_b, r2 + 1, pl.ds(32, 16)]
                a3 = a3 + rows_v[_b, r2 + 1, pl.ds(48, 16)]
                return (a0, a1, a2, a3)

            a0, a1, a2, a3 = lax.fori_loop(0, SEQ // 2, seq_body,
                                           (zeros, zeros, zeros, zeros))
            scale = jnp.float32(1.0 / SEQ)
            pooled_v[row, pl.ds(0, 16)] = a0 * scale
            pooled_v[row, pl.ds(16, 16)] = a1 * scale
            pooled_v[row, pl.ds(32, 16)] = a2 * scale
            pooled_v[row, pl.ds(48, 16)] = a3 * scale
        return carry

    lax.fori_loop(0, _ROWS_PER_W // _NBUF, quad_body, 0)
    pltpu.sync_copy(pooled_v, pooled_hbm.at[pl.ds(base, _ROWS_PER_W)])


_VC = 20000
_VT = VOCAB // _VC
_BT = 128
_KA = 128


_SBT = 512
_SUB = 5000


def _stats_body(w_ref, p_ref, stats_ref, m_scr, s_scr):
    vc = pl.program_id(0)
    bt = pl.program_id(1)
    sl = pl.ds(bt * _SBT, _SBT)
    first = vc == 0
    m_run = jnp.where(first, -jnp.inf, m_scr[:, sl])
    s_run = jnp.where(first, 0.0, s_scr[:, sl])
    p = p_ref[...]
    for off in range(0, _VC, _SUB):
        lt = lax.dot_general(
            w_ref[pl.ds(off, _SUB), :], p,
            dimension_numbers=(((1,), (0,)), ((), ())),
            preferred_element_type=jnp.float32)
        m_new = jnp.maximum(m_run, jnp.max(lt, axis=0, keepdims=True))
        s_run = (s_run * jnp.exp(m_run - m_new)
                 + jnp.sum(jnp.exp(lt - m_new), axis=0, keepdims=True))
        m_run = m_new
    m_scr[:, sl] = m_run
    s_scr[:, sl] = s_run
    stats_ref[0:1, :] = m_run
    stats_ref[1:2, :] = 1.0 / s_run


def _stats(W_aug, pT_aug):
    return pl.pallas_call(
        _stats_body,
        grid=(_VT, BATCH // _SBT),
        in_specs=[
            pl.BlockSpec((_VC, _KA), lambda v, b: (v, 0)),
            pl.BlockSpec((_KA, _SBT), lambda v, b: (0, b)),
        ],
        out_specs=pl.BlockSpec((2, _SBT), lambda v, b: (0, b)),
        out_shape=jax.ShapeDtypeStruct((2, BATCH), jnp.float32),
        scratch_shapes=[
            pltpu.VMEM((1, BATCH), jnp.float32),
            pltpu.VMEM((1, BATCH), jnp.float32),
        ],
    )(W_aug, pT_aug)


def _write_body(w_ref, p_ref, stats_ref, out_ref):
    lt = lax.dot_general(
        w_ref[...], p_ref[...],
        dimension_numbers=(((1,), (0,)), ((), ())),
        preferred_element_type=jnp.float32)
    m = stats_ref[0:1, :]
    inv = stats_ref[1:2, :]
    out_ref[...] = jnp.exp(lt - m) * inv


def _write(W_aug, pT_aug, stats):
    return pl.pallas_call(
        _write_body,
        grid=(_VT, BATCH // _BT),
        in_specs=[
            pl.BlockSpec((_VC, _KA), lambda v, b: (v, 0)),
            pl.BlockSpec((_KA, _BT), lambda v, b: (0, b)),
            pl.BlockSpec((2, _BT), lambda v, b: (0, b)),
        ],
        out_specs=pl.BlockSpec((_VC, _BT), lambda v, b: (v, b)),
        out_shape=jax.ShapeDtypeStruct((VOCAB, BATCH), jnp.float32),
    )(W_aug, pT_aug, stats)


def kernel(x, embed_table, W, b):
    pooled = _pool_sc(x, embed_table)
    W_aug = jnp.concatenate(
        [W.astype(jnp.bfloat16),
         b[:, None].astype(jnp.bfloat16),
         jnp.zeros((VOCAB, _KA - EMBED - 1), jnp.bfloat16)], axis=1)
    pT_aug = jnp.concatenate(
        [pooled.T.astype(jnp.bfloat16),
         jnp.ones((1, BATCH), jnp.bfloat16),
         jnp.zeros((_KA - EMBED - 1, BATCH), jnp.bfloat16)], axis=0)
    stats = _stats(W_aug, pT_aug)
    outT = _write(W_aug, pT_aug, stats)
    return outT.T
